# speedup vs baseline: 1.9905x; 1.9905x over previous
import jax
import jax.numpy as jnp
from jax.experimental import pallas as pl

_EPS = 1e-5


def _mm_kernel(a_ref, b_ref, o_ref):
    o_ref[...] = jnp.dot(a_ref[...], b_ref[...],
                         preferred_element_type=jnp.float32)


def _matmul(a, b, block_rows=512):
    m, k = a.shape
    _, n = b.shape
    return pl.pallas_call(
        _mm_kernel,
        grid=(m // block_rows,),
        in_specs=[
            pl.BlockSpec((block_rows, k), lambda i: (i, 0)),
            pl.BlockSpec((k, n), lambda i: (0, 0)),
        ],
        out_specs=pl.BlockSpec((block_rows, n), lambda i: (i, 0)),
        out_shape=jax.ShapeDtypeStruct((m, n), jnp.float32),
    )(a, b)


def _bn_relu(x, gamma, beta):
    mu = jnp.mean(x, axis=0, keepdims=True)
    xc = x - mu
    var = jnp.mean(xc * xc, axis=0, keepdims=True)
    y = gamma * xc * jax.lax.rsqrt(var + _EPS) + beta
    return jnp.maximum(y, 0.0)


def _tail_kernel(x_ref, w_ref, wn_ref, ga_ref, ba_ref, gb_ref, bb_ref,
                 o_ref):
    r = _bn_relu(x_ref[...], ga_ref[...], ba_ref[...])
    t = jnp.dot(r, w_ref[...], preferred_element_type=jnp.float32)
    h = _bn_relu(t, gb_ref[...], bb_ref[...])
    o_ref[...] = jnp.dot(h, wn_ref[...], preferred_element_type=jnp.float32)


def _tail(x, w, wnext, ga, ba, gb, bb):
    n, h = x.shape
    full = lambda s: pl.BlockSpec(s, lambda: tuple(0 for _ in s))
    return pl.pallas_call(
        _tail_kernel,
        in_specs=[full((n, h)), full((h, h)), full((h, h)),
                  full((1, h)), full((1, h)), full((1, h)), full((1, h))],
        out_specs=full((n, h)),
        out_shape=jax.ShapeDtypeStruct((n, h), jnp.float32),
    )(x, w, wnext, ga.reshape(1, h), ba.reshape(1, h),
      gb.reshape(1, h), bb.reshape(1, h))


def _final_kernel(x_ref, w_ref, ga_ref, ba_ref, gb_ref, bb_ref,
                  nodes_ref, pool_ref):
    r = _bn_relu(x_ref[...], ga_ref[...], ba_ref[...])
    t = jnp.dot(r, w_ref[...], preferred_element_type=jnp.float32)
    h = _bn_relu(t, gb_ref[...], bb_ref[...])
    nodes_ref[...] = h
    pool_ref[...] = jnp.mean(h, axis=0, keepdims=True)


def _final(x, w, ga, ba, gb, bb):
    n, h = x.shape
    full = lambda s: pl.BlockSpec(s, lambda: tuple(0 for _ in s))
    return pl.pallas_call(
        _final_kernel,
        in_specs=[full((n, h)), full((h, h)),
                  full((1, h)), full((1, h)), full((1, h)), full((1, h))],
        out_specs=[full((n, h)), full((1, h))],
        out_shape=[jax.ShapeDtypeStruct((n, h), jnp.float32),
                   jax.ShapeDtypeStruct((1, h), jnp.float32)],
    )(x, w, ga.reshape(1, h), ba.reshape(1, h),
      gb.reshape(1, h), bb.reshape(1, h))


def kernel(features, adjacency_matrix, W0_0, b0_0, g0_0, be0_0, W0_1, b0_1,
           g0, be0, W1_0, b1_0, g1_0, be1_0, W1_1, b1_1, g1, be1):
    x = _matmul(features, W0_0)
    h1 = _matmul(adjacency_matrix, x)
    hw = _tail(h1, W0_1, W1_0, g0_0, be0_0, g0, be0)
    h2 = _matmul(adjacency_matrix, hw)
    nodes, pooled = _final(h2, W1_1, g1_0, be1_0, g1, be1)
    return (pooled, nodes)


# speedup vs baseline: 2.5051x; 1.2586x over previous
import jax
import jax.numpy as jnp
from jax.experimental import pallas as pl
from jax.experimental.pallas import tpu as pltpu

N = 4096
D_IN = 4096
H = 512
_EPS = 1e-5
_BLK = 512
_NBLK = N // _BLK


def _bn_relu(x, gamma, beta):
    mu = jnp.mean(x, axis=0, keepdims=True)
    xc = x - mu
    var = jnp.mean(xc * xc, axis=0, keepdims=True)
    y = gamma * xc * jax.lax.rsqrt(var + _EPS) + beta
    return jnp.maximum(y, 0.0)


def _stream_matmul(hbm_ref, rhs, out_vmem, dbuf, sem):
    def copy(i, slot):
        return pltpu.make_async_copy(
            hbm_ref.at[pl.ds(i * _BLK, _BLK), :], dbuf.at[slot], sem.at[slot])

    copy(0, 0).start()
    for i in range(_NBLK):
        slot = i % 2
        if i + 1 < _NBLK:
            copy(i + 1, (i + 1) % 2).start()
        copy(i, slot).wait()
        out_vmem[pl.ds(i * _BLK, _BLK), :] = jnp.dot(
            dbuf[slot], rhs, preferred_element_type=jnp.float32)


def _gnn_kernel(f_hbm, a_hbm, w00_ref, w01_ref, w10_ref, w11_ref,
                g00_ref, be00_ref, g0_ref, be0_ref,
                g10_ref, be10_ref, g1_ref, be1_ref,
                nodes_ref, pool_ref,
                xv, hv, dbuf, sem):
    _stream_matmul(f_hbm, w00_ref[...], xv, dbuf, sem)
    _stream_matmul(a_hbm, xv[...], hv, dbuf, sem)
    r = _bn_relu(hv[...], g00_ref[...], be00_ref[...])
    t = jnp.dot(r, w01_ref[...], preferred_element_type=jnp.float32)
    h = _bn_relu(t, g0_ref[...], be0_ref[...])
    xv[...] = jnp.dot(h, w10_ref[...], preferred_element_type=jnp.float32)
    _stream_matmul(a_hbm, xv[...], hv, dbuf, sem)
    r = _bn_relu(hv[...], g10_ref[...], be10_ref[...])
    t = jnp.dot(r, w11_ref[...], preferred_element_type=jnp.float32)
    out = _bn_relu(t, g1_ref[...], be1_ref[...])
    nodes_ref[...] = out
    pool_ref[...] = jnp.mean(out, axis=0, keepdims=True)


def kernel(features, adjacency_matrix, W0_0, b0_0, g0_0, be0_0, W0_1, b0_1,
           g0, be0, W1_0, b1_0, g1_0, be1_0, W1_1, b1_1, g1, be1):
    anyspec = pl.BlockSpec(memory_space=pl.ANY)
    full = lambda s: pl.BlockSpec(s, lambda: tuple(0 for _ in s))
    vec = full((1, H))
    nodes, pooled = pl.pallas_call(
        _gnn_kernel,
        in_specs=[anyspec, anyspec,
                  full((D_IN, H)), full((H, H)), full((H, H)), full((H, H)),
                  vec, vec, vec, vec, vec, vec, vec, vec],
        out_specs=[full((N, H)), full((1, H))],
        out_shape=[jax.ShapeDtypeStruct((N, H), jnp.float32),
                   jax.ShapeDtypeStruct((1, H), jnp.float32)],
        scratch_shapes=[
            pltpu.VMEM((N, H), jnp.float32),
            pltpu.VMEM((N, H), jnp.float32),
            pltpu.VMEM((2, _BLK, D_IN), jnp.float32),
            pltpu.SemaphoreType.DMA((2,)),
        ],
    )(features, adjacency_matrix, W0_0, W0_1, W1_0, W1_1,
      g0_0.reshape(1, H), be0_0.reshape(1, H), g0.reshape(1, H),
      be0.reshape(1, H), g1_0.reshape(1, H), be1_0.reshape(1, H),
      g1.reshape(1, H), be1.reshape(1, H))
    return (pooled, nodes)


# speedup vs baseline: 2.5415x; 1.0145x over previous
import jax
import jax.numpy as jnp
from jax.experimental import pallas as pl
from jax.experimental.pallas import tpu as pltpu

N = 4096
D_IN = 4096
H = 512
_EPS = 1e-5
_BLK = 512
_NBLK = N // _BLK


def _bn_relu(x, gamma, beta):
    mu = jnp.mean(x, axis=0, keepdims=True)
    xc = x - mu
    var = jnp.mean(xc * xc, axis=0, keepdims=True)
    y = gamma * xc * jax.lax.rsqrt(var + _EPS) + beta
    return jnp.maximum(y, 0.0)


def _gnn_kernel(f_hbm, a_hbm, w00_ref, w01_ref, w10_ref, w11_ref,
                g00_ref, be00_ref, g0_ref, be0_ref,
                g10_ref, be10_ref, g1_ref, be1_ref,
                nodes_ref, pool_ref,
                xv, hv, dbuf, sem):
    loads = ([(f_hbm, i) for i in range(_NBLK)] +
             [(a_hbm, i) for i in range(_NBLK)] +
             [(a_hbm, i) for i in range(_NBLK)])

    def copy(k, slot):
        src, blk = loads[k]
        return pltpu.make_async_copy(
            src.at[pl.ds(blk * _BLK, _BLK), :], dbuf.at[slot], sem.at[slot])

    copy(0, 0).start()
    for k in range(2 * _NBLK):
        slot = k % 2
        copy(k + 1, (k + 1) % 2).start()
        copy(k, slot).wait()
        rows = pl.ds(loads[k][1] * _BLK, _BLK)
        if k < _NBLK:
            xv[rows, :] = jnp.dot(dbuf[slot], w00_ref[...],
                                  preferred_element_type=jnp.float32)
        else:
            hv[rows, :] = jnp.dot(dbuf[slot], xv[...],
                                  preferred_element_type=jnp.float32)
    r = _bn_relu(hv[...], g00_ref[...], be00_ref[...])
    t = jnp.dot(r, w01_ref[...], preferred_element_type=jnp.float32)
    h = _bn_relu(t, g0_ref[...], be0_ref[...])
    xv[...] = jnp.dot(h, w10_ref[...], preferred_element_type=jnp.float32)
    for k in range(2 * _NBLK, 3 * _NBLK):
        slot = k % 2
        if k + 1 < 3 * _NBLK:
            copy(k + 1, (k + 1) % 2).start()
        copy(k, slot).wait()
        rows = pl.ds(loads[k][1] * _BLK, _BLK)
        hv[rows, :] = jnp.dot(dbuf[slot], xv[...],
                              preferred_element_type=jnp.float32)
    r = _bn_relu(hv[...], g10_ref[...], be10_ref[...])
    t = jnp.dot(r, w11_ref[...], preferred_element_type=jnp.float32)
    out = _bn_relu(t, g1_ref[...], be1_ref[...])
    nodes_ref[...] = out
    pool_ref[...] = jnp.mean(out, axis=0, keepdims=True)


def kernel(features, adjacency_matrix, W0_0, b0_0, g0_0, be0_0, W0_1, b0_1,
           g0, be0, W1_0, b1_0, g1_0, be1_0, W1_1, b1_1, g1, be1):
    anyspec = pl.BlockSpec(memory_space=pl.ANY)
    full = lambda s: pl.BlockSpec(s, lambda: tuple(0 for _ in s))
    vec = full((1, H))
    nodes, pooled = pl.pallas_call(
        _gnn_kernel,
        in_specs=[anyspec, anyspec,
                  full((D_IN, H)), full((H, H)), full((H, H)), full((H, H)),
                  vec, vec, vec, vec, vec, vec, vec, vec],
        out_specs=[full((N, H)), full((1, H))],
        out_shape=[jax.ShapeDtypeStruct((N, H), jnp.float32),
                   jax.ShapeDtypeStruct((1, H), jnp.float32)],
        scratch_shapes=[
            pltpu.VMEM((N, H), jnp.float32),
            pltpu.VMEM((N, H), jnp.float32),
            pltpu.VMEM((2, _BLK, D_IN), jnp.float32),
            pltpu.SemaphoreType.DMA((2,)),
        ],
    )(features, adjacency_matrix, W0_0, W0_1, W1_0, W1_1,
      g0_0.reshape(1, H), be0_0.reshape(1, H), g0.reshape(1, H),
      be0.reshape(1, H), g1_0.reshape(1, H), be1_0.reshape(1, H),
      g1.reshape(1, H), be1.reshape(1, H))
    return (pooled, nodes)


# speedup vs baseline: 2.7718x; 1.0906x over previous
import jax
import jax.numpy as jnp
from jax.experimental import pallas as pl
from jax.experimental.pallas import tpu as pltpu

N = 4096
D_IN = 4096
H = 512
_EPS = 1e-5
_BLK = 256
_NBLK = N // _BLK
_NSLOT = 3
_DEPTH = 2


def _bn_relu(x, gamma, beta):
    mu = jnp.mean(x, axis=0, keepdims=True)
    xc = x - mu
    var = jnp.mean(xc * xc, axis=0, keepdims=True)
    y = gamma * xc * jax.lax.rsqrt(var + _EPS) + beta
    return jnp.maximum(y, 0.0)


def _gnn_kernel(f_hbm, a_hbm, w00_ref, w01_ref, w10_ref, w11_ref,
                g00_ref, be00_ref, g0_ref, be0_ref,
                g10_ref, be10_ref, g1_ref, be1_ref,
                nodes_ref, pool_ref,
                xv, hv, dbuf, sem):
    loads = ([(f_hbm, i) for i in range(_NBLK)] +
             [(a_hbm, i) for i in range(_NBLK)] +
             [(a_hbm, i) for i in range(_NBLK)])

    def copy(k, slot):
        src, blk = loads[k]
        return pltpu.make_async_copy(
            src.at[pl.ds(blk * _BLK, _BLK), :], dbuf.at[slot], sem.at[slot])

    for d in range(_DEPTH):
        copy(d, d % _NSLOT).start()
    for k in range(2 * _NBLK):
        slot = k % _NSLOT
        copy(k + _DEPTH, (k + _DEPTH) % _NSLOT).start()
        copy(k, slot).wait()
        rows = pl.ds(loads[k][1] * _BLK, _BLK)
        if k < _NBLK:
            xv[rows, :] = jnp.dot(dbuf[slot], w00_ref[...],
                                  preferred_element_type=jnp.float32)
        else:
            hv[rows, :] = jnp.dot(dbuf[slot], xv[...],
                                  preferred_element_type=jnp.float32)
    r = _bn_relu(hv[...], g00_ref[...], be00_ref[...])
    t = jnp.dot(r, w01_ref[...], preferred_element_type=jnp.float32)
    h = _bn_relu(t, g0_ref[...], be0_ref[...])
    xv[...] = jnp.dot(h, w10_ref[...], preferred_element_type=jnp.float32)
    for k in range(2 * _NBLK, 3 * _NBLK):
        slot = k % _NSLOT
        if k + _DEPTH < 3 * _NBLK:
            copy(k + _DEPTH, (k + _DEPTH) % _NSLOT).start()
        copy(k, slot).wait()
        rows = pl.ds(loads[k][1] * _BLK, _BLK)
        hv[rows, :] = jnp.dot(dbuf[slot], xv[...],
                              preferred_element_type=jnp.float32)
    r = _bn_relu(hv[...], g10_ref[...], be10_ref[...])
    t = jnp.dot(r, w11_ref[...], preferred_element_type=jnp.float32)
    out = _bn_relu(t, g1_ref[...], be1_ref[...])
    nodes_ref[...] = out
    pool_ref[...] = jnp.mean(out, axis=0, keepdims=True)


def kernel(features, adjacency_matrix, W0_0, b0_0, g0_0, be0_0, W0_1, b0_1,
           g0, be0, W1_0, b1_0, g1_0, be1_0, W1_1, b1_1, g1, be1):
    anyspec = pl.BlockSpec(memory_space=pl.ANY)
    full = lambda s: pl.BlockSpec(s, lambda: tuple(0 for _ in s))
    vec = full((1, H))
    nodes, pooled = pl.pallas_call(
        _gnn_kernel,
        in_specs=[anyspec, anyspec,
                  full((D_IN, H)), full((H, H)), full((H, H)), full((H, H)),
                  vec, vec, vec, vec, vec, vec, vec, vec],
        out_specs=[full((N, H)), full((1, H))],
        out_shape=[jax.ShapeDtypeStruct((N, H), jnp.float32),
                   jax.ShapeDtypeStruct((1, H), jnp.float32)],
        scratch_shapes=[
            pltpu.VMEM((N, H), jnp.float32),
            pltpu.VMEM((N, H), jnp.float32),
            pltpu.VMEM((_NSLOT, _BLK, D_IN), jnp.float32),
            pltpu.SemaphoreType.DMA((_NSLOT,)),
        ],
    )(features, adjacency_matrix, W0_0, W0_1, W1_0, W1_1,
      g0_0.reshape(1, H), be0_0.reshape(1, H), g0.reshape(1, H),
      be0.reshape(1, H), g1_0.reshape(1, H), be1_0.reshape(1, H),
      g1.reshape(1, H), be1.reshape(1, H))
    return (pooled, nodes)
